# baseline (speedup 1.0000x reference)
import functools

import jax
import jax.numpy as jnp
from jax import lax
from jax.experimental import pallas as pl
from jax.experimental.pallas import tpu as pltpu
from jax.experimental.pallas import tpu_sc as plsc

_N = 50000
_NPAD = 51200
_C = 32
_NS = 4
_BATCH = 384
_NW = 32
_G = 132
_NBTOT = _NW * _G
_EPAD = _NBTOT * _BATCH
_ZROWS = 64
_R = 1024


def _sc_agg(table, srcb, dstb):
    mesh = plsc.VectorSubcoreMesh(core_axis_name="c", subcore_axis_name="s")
    rows_per_sub = _NPAD // 16

    @functools.partial(
        pl.kernel,
        out_type=jax.ShapeDtypeStruct((2, _NPAD, _C), jnp.float32),
        mesh=mesh,
        compiler_params=pltpu.CompilerParams(use_tc_tiling_on_sc=False),
        scratch_types=[
            pltpu.VMEM((2, _BATCH), jnp.int32),
            pltpu.VMEM((2, _BATCH), jnp.int32),
            pltpu.VMEM((2, _BATCH, _C), jnp.float32),
            pltpu.VMEM((_ZROWS, _C), jnp.float32),
            pltpu.VMEM_SHARED((_NPAD, _C), jnp.float32),
            pltpu.SemaphoreType.DMA,
            pltpu.SemaphoreType.DMA,
            pltpu.SemaphoreType.DMA,
            pltpu.SemaphoreType.DMA,
        ],
    )
    def k(table_hbm, srcb_hbm, dstb_hbm, out_hbm, src_v, dst_v, rows_v, zbuf,
          acc, gsem0, gsem1, ssem0, ssem1):
        cid = lax.axis_index("c")
        sid = lax.axis_index("s")
        wid = cid * 16 + sid
        gsems = (gsem0, gsem1)
        ssems = (ssem0, ssem1)

        def zb(i, carry):
            zbuf[i, pl.ds(0, 16)] = jnp.zeros((16,), jnp.float32)
            zbuf[i, pl.ds(16, 16)] = jnp.zeros((16,), jnp.float32)
            return carry

        lax.fori_loop(0, _ZROWS, zb, 0)

        def zc(i, carry):
            pltpu.sync_copy(
                zbuf, acc.at[pl.ds(sid * rows_per_sub + i * _ZROWS, _ZROWS)])
            return carry

        lax.fori_loop(0, rows_per_sub // _ZROWS, zc, 0)
        plsc.subcore_barrier()

        base = wid * _G

        def load_idx(g, b):
            pltpu.sync_copy(srcb_hbm.at[base + g], src_v.at[b])
            pltpu.sync_copy(dstb_hbm.at[base + g], dst_v.at[b])

        def fire_gather(b):
            pltpu.async_copy(table_hbm.at[src_v.at[b]], rows_v.at[b],
                             gsems[b])

        def drain_gather(b):
            pltpu.make_async_copy(table_hbm.at[src_v.at[b]], rows_v.at[b],
                                  gsems[b]).wait()

        def fire_scatter(b):
            pltpu.async_copy(rows_v.at[b], acc.at[dst_v.at[b]], ssems[b],
                             add=True)

        def drain_scatter(b):
            pltpu.make_async_copy(rows_v.at[b], acc.at[dst_v.at[b]],
                                  ssems[b]).wait()

        load_idx(0, 0)
        fire_gather(0)

        def body(i, carry):
            g1 = 2 * i + 1

            @pl.when(i > 0)
            def _():
                drain_scatter(1)

            load_idx(g1, 1)
            fire_gather(1)
            drain_gather(0)
            fire_scatter(0)
            drain_scatter(0)

            @pl.when(g1 + 1 < _G)
            def _():
                load_idx(g1 + 1, 0)
                fire_gather(0)

            drain_gather(1)
            fire_scatter(1)
            return carry

        lax.fori_loop(0, _G // 2, body, 0)
        drain_scatter(1)
        plsc.subcore_barrier()

        def wb(i, carry):
            off = sid * rows_per_sub + i * 640
            pltpu.sync_copy(acc.at[pl.ds(off, 640)],
                            out_hbm.at[cid].at[pl.ds(off, 640)])
            return carry

        lax.fori_loop(0, rows_per_sub // 640, wb, 0)

    return k(table, srcb, dstb)


def _tc_layer1(part, x_pad, wlt, wrt, b):
    def body(p_ref, x_ref, wl_ref, wr_ref, b_ref, s0, s1, s2, s3, inv_ref):
        p = p_ref[0] + p_ref[1]
        cnt = p[:, 26:27]
        inv = 1.0 / jnp.maximum(cnt, 1.0)
        h = (inv * jnp.dot(p, wl_ref[...], preferred_element_type=jnp.float32)
             + jnp.dot(x_ref[...], wr_ref[...],
                       preferred_element_type=jnp.float32)
             + b_ref[...])
        h = jnp.maximum(h, 0.0)
        s0[...] = h[:, 0:32]
        s1[...] = h[:, 32:64]
        s2[...] = h[:, 64:96]
        s3[...] = h[:, 96:128]
        inv_ref[...] = inv

    slab = jax.ShapeDtypeStruct((_NPAD, _C), jnp.float32)
    return pl.pallas_call(
        body,
        grid=(_NPAD // _R,),
        in_specs=[
            pl.BlockSpec((2, _R, _C), lambda i: (0, i, 0)),
            pl.BlockSpec((_R, _C), lambda i: (i, 0)),
            pl.BlockSpec((_C, 128), lambda i: (0, 0)),
            pl.BlockSpec((_C, 128), lambda i: (0, 0)),
            pl.BlockSpec((1, 128), lambda i: (0, 0)),
        ],
        out_specs=[
            pl.BlockSpec((_R, _C), lambda i: (i, 0)),
            pl.BlockSpec((_R, _C), lambda i: (i, 0)),
            pl.BlockSpec((_R, _C), lambda i: (i, 0)),
            pl.BlockSpec((_R, _C), lambda i: (i, 0)),
            pl.BlockSpec((_R, 1), lambda i: (i, 0)),
        ],
        out_shape=[slab, slab, slab, slab,
                   jax.ShapeDtypeStruct((_NPAD, 1), jnp.float32)],
    )(part, x_pad, wlt, wrt, b)


def _tc_layer(parts, slabs, inv, wlt, wrt, b, final):
    def body(p0, p1, p2, p3, s0, s1, s2, s3, inv_ref, wl_ref, wr_ref, b_ref,
             *outs):
        mm = b_ref[...] + jnp.zeros((_R, 128), jnp.float32)
        agg_mm = jnp.zeros((_R, 128), jnp.float32)
        for c, (p_ref, s_ref) in enumerate(
                zip((p0, p1, p2, p3), (s0, s1, s2, s3))):
            agg_mm = agg_mm + jnp.dot(p_ref[0] + p_ref[1],
                                      wl_ref[pl.ds(c * _C, _C), :],
                                      preferred_element_type=jnp.float32)
            mm = mm + jnp.dot(s_ref[...], wr_ref[pl.ds(c * _C, _C), :],
                              preferred_element_type=jnp.float32)
        h = jnp.maximum(inv_ref[...] * agg_mm + mm, 0.0)
        if final:
            outs[0][...] = h
        else:
            for c in range(_NS):
                outs[c][...] = h[:, c * _C:(c + 1) * _C]

    part_spec = pl.BlockSpec((2, _R, _C), lambda i: (0, i, 0))
    slab_spec = pl.BlockSpec((_R, _C), lambda i: (i, 0))
    w_spec = pl.BlockSpec((128, 128), lambda i: (0, 0))
    if final:
        grid = ((_N + _R - 1) // _R,)
        out_specs = [pl.BlockSpec((_R, 128), lambda i: (i, 0))]
        out_shape = [jax.ShapeDtypeStruct((_N, 128), jnp.float32)]
    else:
        grid = (_NPAD // _R,)
        out_specs = [slab_spec] * _NS
        out_shape = [jax.ShapeDtypeStruct((_NPAD, _C), jnp.float32)] * _NS
    out = pl.pallas_call(
        body,
        grid=grid,
        in_specs=[part_spec] * _NS + [slab_spec] * _NS + [
            pl.BlockSpec((_R, 1), lambda i: (i, 0)), w_spec, w_spec,
            pl.BlockSpec((1, 128), lambda i: (0, 0)),
        ],
        out_specs=out_specs,
        out_shape=out_shape,
    )(*parts, *slabs, inv, wlt, wrt, b)
    return out[0] if final else out


def kernel(x, edge_index, W1l, W1r, b1, W2l, W2r, b2, W3l, W3r, b3):
    e = edge_index.shape[1]
    src = edge_index[0].astype(jnp.int32)
    dst = edge_index[1].astype(jnp.int32)
    pad = jnp.full((_EPAD - e,), _N, jnp.int32)
    srcb = jnp.concatenate([src, pad]).reshape(_NBTOT, _BATCH)
    dstb = jnp.concatenate([dst, pad]).reshape(_NBTOT, _BATCH)

    x_pad = jnp.zeros((_NPAD, _C), jnp.float32)
    x_pad = x_pad.at[:_N, :26].set(x).at[:_N, 26].set(1.0)

    w1lt = jnp.zeros((_C, 128), jnp.float32).at[:26].set(W1l.T)
    w1rt = jnp.zeros((_C, 128), jnp.float32).at[:26].set(W1r.T)

    part1 = _sc_agg(x_pad, srcb, dstb)
    *slabs1, inv = _tc_layer1(part1, x_pad, w1lt, w1rt, b1.reshape(1, 128))

    parts2 = [_sc_agg(slabs1[c], srcb, dstb) for c in range(_NS)]
    slabs2 = _tc_layer(parts2, slabs1, inv, W2l.T, W2r.T,
                       b2.reshape(1, 128), final=False)

    parts3 = [_sc_agg(slabs2[c], srcb, dstb) for c in range(_NS)]
    return _tc_layer(parts3, slabs2, inv, W3l.T, W3r.T,
                     b3.reshape(1, 128), final=True)


# speedup vs baseline: 1.0001x; 1.0001x over previous
import functools

import jax
import jax.numpy as jnp
from jax import lax
from jax.experimental import pallas as pl
from jax.experimental.pallas import tpu as pltpu
from jax.experimental.pallas import tpu_sc as plsc

_N = 50000
_NPAD = 51200
_C = 32
_NS = 4
_BATCH = 256
_NW = 32
_G = 198
_NBTOT = _NW * _G
_EPAD = _NBTOT * _BATCH
_ZROWS = 64
_R = 1024


def _sc_agg(table, srcb, dstb):
    mesh = plsc.VectorSubcoreMesh(core_axis_name="c", subcore_axis_name="s")
    rows_per_sub = _NPAD // 16

    @functools.partial(
        pl.kernel,
        out_type=jax.ShapeDtypeStruct((2, _NPAD, _C), jnp.float32),
        mesh=mesh,
        compiler_params=pltpu.CompilerParams(use_tc_tiling_on_sc=False),
        scratch_types=[
            pltpu.VMEM((3, _BATCH), jnp.int32),
            pltpu.VMEM((3, _BATCH), jnp.int32),
            pltpu.VMEM((3, _BATCH, _C), jnp.float32),
            pltpu.VMEM((_ZROWS, _C), jnp.float32),
            pltpu.VMEM_SHARED((_NPAD, _C), jnp.float32),
            pltpu.SemaphoreType.DMA,
            pltpu.SemaphoreType.DMA,
            pltpu.SemaphoreType.DMA,
            pltpu.SemaphoreType.DMA,
            pltpu.SemaphoreType.DMA,
            pltpu.SemaphoreType.DMA,
        ],
    )
    def k(table_hbm, srcb_hbm, dstb_hbm, out_hbm, src_v, dst_v, rows_v, zbuf,
          acc, gsem0, gsem1, gsem2, ssem0, ssem1, ssem2):
        cid = lax.axis_index("c")
        sid = lax.axis_index("s")
        wid = cid * 16 + sid
        gsems = (gsem0, gsem1, gsem2)
        ssems = (ssem0, ssem1, ssem2)

        def zb(i, carry):
            zbuf[i, pl.ds(0, 16)] = jnp.zeros((16,), jnp.float32)
            zbuf[i, pl.ds(16, 16)] = jnp.zeros((16,), jnp.float32)
            return carry

        lax.fori_loop(0, _ZROWS, zb, 0)

        def zc(i, carry):
            pltpu.sync_copy(
                zbuf, acc.at[pl.ds(sid * rows_per_sub + i * _ZROWS, _ZROWS)])
            return carry

        lax.fori_loop(0, rows_per_sub // _ZROWS, zc, 0)
        plsc.subcore_barrier()

        base = wid * _G

        def load_idx(g, b):
            pltpu.sync_copy(srcb_hbm.at[base + g], src_v.at[b])
            pltpu.sync_copy(dstb_hbm.at[base + g], dst_v.at[b])

        def fire_gather(b):
            pltpu.async_copy(table_hbm.at[src_v.at[b]], rows_v.at[b],
                             gsems[b])

        def drain_gather(b):
            pltpu.make_async_copy(table_hbm.at[src_v.at[b]], rows_v.at[b],
                                  gsems[b]).wait()

        def fire_scatter(b):
            pltpu.async_copy(rows_v.at[b], acc.at[dst_v.at[b]], ssems[b],
                             add=True)

        def drain_scatter(b):
            pltpu.make_async_copy(rows_v.at[b], acc.at[dst_v.at[b]],
                                  ssems[b]).wait()

        load_idx(0, 0)
        fire_gather(0)
        load_idx(1, 1)
        fire_gather(1)

        def body(i, carry):
            for j in range(3):
                g = 3 * i + j
                bb = (j + 2) % 3
                drain_gather(j)
                fire_scatter(j)
                if j == 0:
                    @pl.when(i > 0)
                    def _():
                        drain_scatter(bb)
                else:
                    drain_scatter(bb)

                @pl.when(g + 2 < _G)
                def _():
                    load_idx(g + 2, bb)
                    fire_gather(bb)
            return carry

        lax.fori_loop(0, _G // 3, body, 0)
        drain_scatter(2)
        plsc.subcore_barrier()

        def wb(i, carry):
            off = sid * rows_per_sub + i * 640
            pltpu.sync_copy(acc.at[pl.ds(off, 640)],
                            out_hbm.at[cid].at[pl.ds(off, 640)])
            return carry

        lax.fori_loop(0, rows_per_sub // 640, wb, 0)

    return k(table, srcb, dstb)


def _tc_layer1(part, x_pad, wlt, wrt, b):
    def body(p_ref, x_ref, wl_ref, wr_ref, b_ref, s0, s1, s2, s3, inv_ref):
        p = p_ref[0] + p_ref[1]
        cnt = p[:, 26:27]
        inv = 1.0 / jnp.maximum(cnt, 1.0)
        h = (inv * jnp.dot(p, wl_ref[...], preferred_element_type=jnp.float32)
             + jnp.dot(x_ref[...], wr_ref[...],
                       preferred_element_type=jnp.float32)
             + b_ref[...])
        h = jnp.maximum(h, 0.0)
        s0[...] = h[:, 0:32]
        s1[...] = h[:, 32:64]
        s2[...] = h[:, 64:96]
        s3[...] = h[:, 96:128]
        inv_ref[...] = inv

    slab = jax.ShapeDtypeStruct((_NPAD, _C), jnp.float32)
    return pl.pallas_call(
        body,
        grid=(_NPAD // _R,),
        in_specs=[
            pl.BlockSpec((2, _R, _C), lambda i: (0, i, 0)),
            pl.BlockSpec((_R, _C), lambda i: (i, 0)),
            pl.BlockSpec((_C, 128), lambda i: (0, 0)),
            pl.BlockSpec((_C, 128), lambda i: (0, 0)),
            pl.BlockSpec((1, 128), lambda i: (0, 0)),
        ],
        out_specs=[
            pl.BlockSpec((_R, _C), lambda i: (i, 0)),
            pl.BlockSpec((_R, _C), lambda i: (i, 0)),
            pl.BlockSpec((_R, _C), lambda i: (i, 0)),
            pl.BlockSpec((_R, _C), lambda i: (i, 0)),
            pl.BlockSpec((_R, 1), lambda i: (i, 0)),
        ],
        out_shape=[slab, slab, slab, slab,
                   jax.ShapeDtypeStruct((_NPAD, 1), jnp.float32)],
    )(part, x_pad, wlt, wrt, b)


def _tc_layer(parts, slabs, inv, wlt, wrt, b, final):
    def body(p0, p1, p2, p3, s0, s1, s2, s3, inv_ref, wl_ref, wr_ref, b_ref,
             *outs):
        mm = b_ref[...] + jnp.zeros((_R, 128), jnp.float32)
        agg_mm = jnp.zeros((_R, 128), jnp.float32)
        for c, (p_ref, s_ref) in enumerate(
                zip((p0, p1, p2, p3), (s0, s1, s2, s3))):
            agg_mm = agg_mm + jnp.dot(p_ref[0] + p_ref[1],
                                      wl_ref[pl.ds(c * _C, _C), :],
                                      preferred_element_type=jnp.float32)
            mm = mm + jnp.dot(s_ref[...], wr_ref[pl.ds(c * _C, _C), :],
                              preferred_element_type=jnp.float32)
        h = jnp.maximum(inv_ref[...] * agg_mm + mm, 0.0)
        if final:
            outs[0][...] = h
        else:
            for c in range(_NS):
                outs[c][...] = h[:, c * _C:(c + 1) * _C]

    part_spec = pl.BlockSpec((2, _R, _C), lambda i: (0, i, 0))
    slab_spec = pl.BlockSpec((_R, _C), lambda i: (i, 0))
    w_spec = pl.BlockSpec((128, 128), lambda i: (0, 0))
    if final:
        grid = ((_N + _R - 1) // _R,)
        out_specs = [pl.BlockSpec((_R, 128), lambda i: (i, 0))]
        out_shape = [jax.ShapeDtypeStruct((_N, 128), jnp.float32)]
    else:
        grid = (_NPAD // _R,)
        out_specs = [slab_spec] * _NS
        out_shape = [jax.ShapeDtypeStruct((_NPAD, _C), jnp.float32)] * _NS
    out = pl.pallas_call(
        body,
        grid=grid,
        in_specs=[part_spec] * _NS + [slab_spec] * _NS + [
            pl.BlockSpec((_R, 1), lambda i: (i, 0)), w_spec, w_spec,
            pl.BlockSpec((1, 128), lambda i: (0, 0)),
        ],
        out_specs=out_specs,
        out_shape=out_shape,
    )(*parts, *slabs, inv, wlt, wrt, b)
    return out[0] if final else out


def kernel(x, edge_index, W1l, W1r, b1, W2l, W2r, b2, W3l, W3r, b3):
    e = edge_index.shape[1]
    src = edge_index[0].astype(jnp.int32)
    dst = edge_index[1].astype(jnp.int32)
    pad = jnp.full((_EPAD - e,), _N, jnp.int32)
    srcb = jnp.concatenate([src, pad]).reshape(_NBTOT, _BATCH)
    dstb = jnp.concatenate([dst, pad]).reshape(_NBTOT, _BATCH)

    x_pad = jnp.zeros((_NPAD, _C), jnp.float32)
    x_pad = x_pad.at[:_N, :26].set(x).at[:_N, 26].set(1.0)

    w1lt = jnp.zeros((_C, 128), jnp.float32).at[:26].set(W1l.T)
    w1rt = jnp.zeros((_C, 128), jnp.float32).at[:26].set(W1r.T)

    part1 = _sc_agg(x_pad, srcb, dstb)
    *slabs1, inv = _tc_layer1(part1, x_pad, w1lt, w1rt, b1.reshape(1, 128))

    parts2 = [_sc_agg(slabs1[c], srcb, dstb) for c in range(_NS)]
    slabs2 = _tc_layer(parts2, slabs1, inv, W2l.T, W2r.T,
                       b2.reshape(1, 128), final=False)

    parts3 = [_sc_agg(slabs2[c], srcb, dstb) for c in range(_NS)]
    return _tc_layer(parts3, slabs2, inv, W3l.T, W3r.T,
                     b3.reshape(1, 128), final=True)


# speedup vs baseline: 1.2676x; 1.2675x over previous
import functools

import jax
import jax.numpy as jnp
from jax import lax
from jax.experimental import pallas as pl
from jax.experimental.pallas import tpu as pltpu
from jax.experimental.pallas import tpu_sc as plsc

_N = 50000
_NPAD = 51200
_C = 32
_NS = 4
_BATCH = 256
_NW = 32
_G = 196
_NBTOT = _NW * _G
_EPAD = _NBTOT * _BATCH
_ZROWS = 64
_R = 1024


def _sc_agg(table, srcb, dstb):
    mesh = plsc.VectorSubcoreMesh(core_axis_name="c", subcore_axis_name="s")
    rows_per_sub = _NPAD // 16

    @functools.partial(
        pl.kernel,
        out_type=jax.ShapeDtypeStruct((2, _NPAD, _C), jnp.float32),
        mesh=mesh,
        compiler_params=pltpu.CompilerParams(use_tc_tiling_on_sc=False),
        scratch_types=[
            pltpu.VMEM((2, _BATCH), jnp.int32),
            pltpu.VMEM((2, _BATCH), jnp.int32),
            pltpu.VMEM((2, _BATCH, _C), jnp.float32),
            pltpu.VMEM((_ZROWS, _C), jnp.float32),
            pltpu.VMEM_SHARED((_NPAD, _C), jnp.float32),
            pltpu.SemaphoreType.DMA,
            pltpu.SemaphoreType.DMA,
            pltpu.SemaphoreType.DMA,
            pltpu.SemaphoreType.DMA,
            pltpu.SemaphoreType.DMA,
        ],
    )
    def k(table_hbm, srcb_hbm, dstb_hbm, out_hbm, src_v, dst_v, rows_v, zbuf,
          acc, gsem0, gsem1, ssem0, ssem1, zsem):
        cid = lax.axis_index("c")
        sid = lax.axis_index("s")
        wid = cid * 16 + sid
        gsems = (gsem0, gsem1)
        ssems = (ssem0, ssem1)

        base = wid * _G

        def load_idx(g, b):
            pltpu.sync_copy(srcb_hbm.at[base + g], src_v.at[b])
            pltpu.sync_copy(dstb_hbm.at[base + g], dst_v.at[b])

        def fire_gather(b):
            pltpu.async_copy(table_hbm.at[src_v.at[b]], rows_v.at[b],
                             gsems[b])

        def drain_gather(b):
            pltpu.make_async_copy(table_hbm.at[src_v.at[b]], rows_v.at[b],
                                  gsems[b]).wait()

        def fire_scatter(b):
            pltpu.async_copy(rows_v.at[b], acc.at[dst_v.at[b]], ssems[b],
                             add=True)

        def drain_scatter(b):
            pltpu.make_async_copy(rows_v.at[b], acc.at[dst_v.at[b]],
                                  ssems[b]).wait()

        load_idx(0, 0)
        fire_gather(0)
        load_idx(1, 1)
        fire_gather(1)

        def zb(i, carry):
            zbuf[i, pl.ds(0, 16)] = jnp.zeros((16,), jnp.float32)
            zbuf[i, pl.ds(16, 16)] = jnp.zeros((16,), jnp.float32)
            return carry

        lax.fori_loop(0, _ZROWS, zb, 0)
        nz = rows_per_sub // _ZROWS

        def zc(i, carry):
            pltpu.async_copy(
                zbuf, acc.at[pl.ds(sid * rows_per_sub + i * _ZROWS, _ZROWS)],
                zsem)
            return carry

        lax.fori_loop(0, nz, zc, 0)

        def zw(i, carry):
            pltpu.make_async_copy(
                zbuf, acc.at[pl.ds(sid * rows_per_sub, _ZROWS)], zsem).wait()
            return carry

        lax.fori_loop(0, nz, zw, 0)
        plsc.subcore_barrier()

        def body(i, carry):
            g1 = 2 * i + 1

            @pl.when(i > 0)
            def _():
                drain_scatter(1)

            @pl.when(i > 0)
            def _():
                load_idx(g1, 1)
                fire_gather(1)

            drain_gather(0)
            fire_scatter(0)
            drain_scatter(0)

            @pl.when(g1 + 1 < _G)
            def _():
                load_idx(g1 + 1, 0)
                fire_gather(0)

            drain_gather(1)
            fire_scatter(1)
            return carry

        lax.fori_loop(0, _G // 2, body, 0)
        drain_scatter(1)
        plsc.subcore_barrier()

        nwb = rows_per_sub // 640

        def wb(i, carry):
            off = sid * rows_per_sub + i * 640
            pltpu.async_copy(acc.at[pl.ds(off, 640)],
                             out_hbm.at[cid].at[pl.ds(off, 640)], zsem)
            return carry

        lax.fori_loop(0, nwb, wb, 0)

        def wbw(i, carry):
            off = sid * rows_per_sub + i * 640
            pltpu.make_async_copy(acc.at[pl.ds(off, 640)],
                                  out_hbm.at[cid].at[pl.ds(off, 640)],
                                  zsem).wait()
            return carry

        lax.fori_loop(0, nwb, wbw, 0)

    return k(table, srcb, dstb)


def _tc_layer1(part, x_pad, wlt, wrt, b):
    def body(p_ref, x_ref, wl_ref, wr_ref, b_ref, s0, s1, s2, s3, inv_ref):
        p = p_ref[0] + p_ref[1]
        cnt = p[:, 26:27]
        inv = 1.0 / jnp.maximum(cnt, 1.0)
        h = (inv * jnp.dot(p, wl_ref[...], preferred_element_type=jnp.float32)
             + jnp.dot(x_ref[...], wr_ref[...],
                       preferred_element_type=jnp.float32)
             + b_ref[...])
        h = jnp.maximum(h, 0.0)
        s0[...] = h[:, 0:32]
        s1[...] = h[:, 32:64]
        s2[...] = h[:, 64:96]
        s3[...] = h[:, 96:128]
        inv_ref[...] = inv

    slab = jax.ShapeDtypeStruct((_NPAD, _C), jnp.float32)
    return pl.pallas_call(
        body,
        grid=(_NPAD // _R,),
        in_specs=[
            pl.BlockSpec((2, _R, _C), lambda i: (0, i, 0)),
            pl.BlockSpec((_R, _C), lambda i: (i, 0)),
            pl.BlockSpec((_C, 128), lambda i: (0, 0)),
            pl.BlockSpec((_C, 128), lambda i: (0, 0)),
            pl.BlockSpec((1, 128), lambda i: (0, 0)),
        ],
        out_specs=[
            pl.BlockSpec((_R, _C), lambda i: (i, 0)),
            pl.BlockSpec((_R, _C), lambda i: (i, 0)),
            pl.BlockSpec((_R, _C), lambda i: (i, 0)),
            pl.BlockSpec((_R, _C), lambda i: (i, 0)),
            pl.BlockSpec((_R, 1), lambda i: (i, 0)),
        ],
        out_shape=[slab, slab, slab, slab,
                   jax.ShapeDtypeStruct((_NPAD, 1), jnp.float32)],
    )(part, x_pad, wlt, wrt, b)


def _tc_layer(parts, slabs, inv, wlt, wrt, b, final):
    def body(p0, p1, p2, p3, s0, s1, s2, s3, inv_ref, wl_ref, wr_ref, b_ref,
             *outs):
        mm = b_ref[...] + jnp.zeros((_R, 128), jnp.float32)
        agg_mm = jnp.zeros((_R, 128), jnp.float32)
        for c, (p_ref, s_ref) in enumerate(
                zip((p0, p1, p2, p3), (s0, s1, s2, s3))):
            agg_mm = agg_mm + jnp.dot(p_ref[0] + p_ref[1],
                                      wl_ref[pl.ds(c * _C, _C), :],
                                      preferred_element_type=jnp.float32)
            mm = mm + jnp.dot(s_ref[...], wr_ref[pl.ds(c * _C, _C), :],
                              preferred_element_type=jnp.float32)
        h = jnp.maximum(inv_ref[...] * agg_mm + mm, 0.0)
        if final:
            outs[0][...] = h
        else:
            for c in range(_NS):
                outs[c][...] = h[:, c * _C:(c + 1) * _C]

    part_spec = pl.BlockSpec((2, _R, _C), lambda i: (0, i, 0))
    slab_spec = pl.BlockSpec((_R, _C), lambda i: (i, 0))
    w_spec = pl.BlockSpec((128, 128), lambda i: (0, 0))
    if final:
        grid = ((_N + _R - 1) // _R,)
        out_specs = [pl.BlockSpec((_R, 128), lambda i: (i, 0))]
        out_shape = [jax.ShapeDtypeStruct((_N, 128), jnp.float32)]
    else:
        grid = (_NPAD // _R,)
        out_specs = [slab_spec] * _NS
        out_shape = [jax.ShapeDtypeStruct((_NPAD, _C), jnp.float32)] * _NS
    out = pl.pallas_call(
        body,
        grid=grid,
        in_specs=[part_spec] * _NS + [slab_spec] * _NS + [
            pl.BlockSpec((_R, 1), lambda i: (i, 0)), w_spec, w_spec,
            pl.BlockSpec((1, 128), lambda i: (0, 0)),
        ],
        out_specs=out_specs,
        out_shape=out_shape,
    )(*parts, *slabs, inv, wlt, wrt, b)
    return out[0] if final else out


def kernel(x, edge_index, W1l, W1r, b1, W2l, W2r, b2, W3l, W3r, b3):
    e = edge_index.shape[1]
    src = edge_index[0].astype(jnp.int32)
    dst = edge_index[1].astype(jnp.int32)
    pad = jnp.full((_EPAD - e,), _N, jnp.int32)
    srcb = jnp.concatenate([src, pad]).reshape(_NBTOT, _BATCH)
    dstb = jnp.concatenate([dst, pad]).reshape(_NBTOT, _BATCH)

    x_pad = jnp.zeros((_NPAD, _C), jnp.float32)
    x_pad = x_pad.at[:_N, :26].set(x).at[:_N, 26].set(1.0)

    w1lt = jnp.zeros((_C, 128), jnp.float32).at[:26].set(W1l.T)
    w1rt = jnp.zeros((_C, 128), jnp.float32).at[:26].set(W1r.T)

    part1 = _sc_agg(x_pad, srcb, dstb)
    *slabs1, inv = _tc_layer1(part1, x_pad, w1lt, w1rt, b1.reshape(1, 128))

    parts2 = [_sc_agg(slabs1[c], srcb, dstb) for c in range(_NS)]
    slabs2 = _tc_layer(parts2, slabs1, inv, W2l.T, W2r.T,
                       b2.reshape(1, 128), final=False)

    parts3 = [_sc_agg(slabs2[c], srcb, dstb) for c in range(_NS)]
    return _tc_layer(parts3, slabs2, inv, W3l.T, W3r.T,
                     b3.reshape(1, 128), final=True)


# speedup vs baseline: 1.5153x; 1.1954x over previous
import functools

import jax
import jax.numpy as jnp
from jax import lax
from jax.experimental import pallas as pl
from jax.experimental.pallas import tpu as pltpu
from jax.experimental.pallas import tpu_sc as plsc

_N = 50000
_NPAD = 51200
_C = 32
_NS = 4
_BATCH = 256
_NW = 32
_G = 196
_NBTOT = _NW * _G
_EPAD = _NBTOT * _BATCH
_ZROWS = 64
_R = 1024


def _sc_agg(table, edgb):
    mesh = plsc.VectorSubcoreMesh(core_axis_name="c", subcore_axis_name="s")
    rows_per_sub = _NPAD // 16

    @functools.partial(
        pl.kernel,
        out_type=jax.ShapeDtypeStruct((2, _NPAD, _C), jnp.float32),
        mesh=mesh,
        compiler_params=pltpu.CompilerParams(use_tc_tiling_on_sc=False),
        scratch_types=[
            pltpu.VMEM((2, 2, _BATCH), jnp.int32),
            pltpu.VMEM((2, _BATCH, _C), jnp.float32),
            pltpu.VMEM((_ZROWS, _C), jnp.float32),
            pltpu.VMEM_SHARED((_NPAD, _C), jnp.float32),
            pltpu.SemaphoreType.DMA,
            pltpu.SemaphoreType.DMA,
            pltpu.SemaphoreType.DMA,
            pltpu.SemaphoreType.DMA,
            pltpu.SemaphoreType.DMA,
        ],
    )
    def k(table_hbm, edgb_hbm, out_hbm, ed_v, rows_v, zbuf,
          acc, gsem0, gsem1, ssem0, ssem1, zsem):
        cid = lax.axis_index("c")
        sid = lax.axis_index("s")
        wid = cid * 16 + sid
        gsems = (gsem0, gsem1)
        ssems = (ssem0, ssem1)

        base = wid * _G

        def load_idx(g, b):
            pltpu.sync_copy(edgb_hbm.at[base + g], ed_v.at[b])

        def fire_gather(b):
            pltpu.async_copy(table_hbm.at[ed_v.at[b, 0]], rows_v.at[b],
                             gsems[b])

        def drain_gather(b):
            pltpu.make_async_copy(table_hbm.at[ed_v.at[b, 0]], rows_v.at[b],
                                  gsems[b]).wait()

        def fire_scatter(b):
            pltpu.async_copy(rows_v.at[b], acc.at[ed_v.at[b, 1]], ssems[b],
                             add=True)

        def drain_scatter(b):
            pltpu.make_async_copy(rows_v.at[b], acc.at[ed_v.at[b, 1]],
                                  ssems[b]).wait()

        load_idx(0, 0)
        fire_gather(0)
        load_idx(1, 1)
        fire_gather(1)

        def zb(i, carry):
            zbuf[i, pl.ds(0, 16)] = jnp.zeros((16,), jnp.float32)
            zbuf[i, pl.ds(16, 16)] = jnp.zeros((16,), jnp.float32)
            return carry

        lax.fori_loop(0, _ZROWS, zb, 0)
        nz = rows_per_sub // _ZROWS

        def zc(i, carry):
            pltpu.async_copy(
                zbuf, acc.at[pl.ds(sid * rows_per_sub + i * _ZROWS, _ZROWS)],
                zsem)
            return carry

        lax.fori_loop(0, nz, zc, 0)

        def zw(i, carry):
            pltpu.make_async_copy(
                zbuf, acc.at[pl.ds(sid * rows_per_sub, _ZROWS)], zsem).wait()
            return carry

        lax.fori_loop(0, nz, zw, 0)
        plsc.subcore_barrier()

        def body(i, carry):
            g1 = 2 * i + 1

            @pl.when(i > 0)
            def _():
                drain_scatter(1)

            @pl.when(i > 0)
            def _():
                load_idx(g1, 1)
                fire_gather(1)

            drain_gather(0)
            fire_scatter(0)
            drain_scatter(0)

            @pl.when(g1 + 1 < _G)
            def _():
                load_idx(g1 + 1, 0)
                fire_gather(0)

            drain_gather(1)
            fire_scatter(1)
            return carry

        lax.fori_loop(0, _G // 2, body, 0)
        drain_scatter(1)
        plsc.subcore_barrier()

        nwb = rows_per_sub // 640

        def wb(i, carry):
            off = sid * rows_per_sub + i * 640
            pltpu.async_copy(acc.at[pl.ds(off, 640)],
                             out_hbm.at[cid].at[pl.ds(off, 640)], zsem)
            return carry

        lax.fori_loop(0, nwb, wb, 0)

        def wbw(i, carry):
            off = sid * rows_per_sub + i * 640
            pltpu.make_async_copy(acc.at[pl.ds(off, 640)],
                                  out_hbm.at[cid].at[pl.ds(off, 640)],
                                  zsem).wait()
            return carry

        lax.fori_loop(0, nwb, wbw, 0)

    return k(table, edgb)


def _tc_layer1(part, x_pad, wlt, wrt, b):
    def body(p_ref, x_ref, wl_ref, wr_ref, b_ref, s0, s1, s2, s3, inv_ref):
        p = p_ref[0] + p_ref[1]
        cnt = p[:, 26:27]
        inv = 1.0 / jnp.maximum(cnt, 1.0)
        h = (inv * jnp.dot(p, wl_ref[...], preferred_element_type=jnp.float32)
             + jnp.dot(x_ref[...], wr_ref[...],
                       preferred_element_type=jnp.float32)
             + b_ref[...])
        h = jnp.maximum(h, 0.0)
        s0[...] = h[:, 0:32]
        s1[...] = h[:, 32:64]
        s2[...] = h[:, 64:96]
        s3[...] = h[:, 96:128]
        inv_ref[...] = inv

    slab = jax.ShapeDtypeStruct((_NPAD, _C), jnp.float32)
    return pl.pallas_call(
        body,
        grid=(_NPAD // _R,),
        in_specs=[
            pl.BlockSpec((2, _R, _C), lambda i: (0, i, 0)),
            pl.BlockSpec((_R, _C), lambda i: (i, 0)),
            pl.BlockSpec((_C, 128), lambda i: (0, 0)),
            pl.BlockSpec((_C, 128), lambda i: (0, 0)),
            pl.BlockSpec((1, 128), lambda i: (0, 0)),
        ],
        out_specs=[
            pl.BlockSpec((_R, _C), lambda i: (i, 0)),
            pl.BlockSpec((_R, _C), lambda i: (i, 0)),
            pl.BlockSpec((_R, _C), lambda i: (i, 0)),
            pl.BlockSpec((_R, _C), lambda i: (i, 0)),
            pl.BlockSpec((_R, 1), lambda i: (i, 0)),
        ],
        out_shape=[slab, slab, slab, slab,
                   jax.ShapeDtypeStruct((_NPAD, 1), jnp.float32)],
    )(part, x_pad, wlt, wrt, b)


def _tc_layer(parts, slabs, inv, wlt, wrt, b, final):
    def body(p0, p1, p2, p3, s0, s1, s2, s3, inv_ref, wl_ref, wr_ref, b_ref,
             *outs):
        mm = b_ref[...] + jnp.zeros((_R, 128), jnp.float32)
        agg_mm = jnp.zeros((_R, 128), jnp.float32)
        for c, (p_ref, s_ref) in enumerate(
                zip((p0, p1, p2, p3), (s0, s1, s2, s3))):
            agg_mm = agg_mm + jnp.dot(p_ref[0] + p_ref[1],
                                      wl_ref[pl.ds(c * _C, _C), :],
                                      preferred_element_type=jnp.float32)
            mm = mm + jnp.dot(s_ref[...], wr_ref[pl.ds(c * _C, _C), :],
                              preferred_element_type=jnp.float32)
        h = jnp.maximum(inv_ref[...] * agg_mm + mm, 0.0)
        if final:
            outs[0][...] = h
        else:
            for c in range(_NS):
                outs[c][...] = h[:, c * _C:(c + 1) * _C]

    part_spec = pl.BlockSpec((2, _R, _C), lambda i: (0, i, 0))
    slab_spec = pl.BlockSpec((_R, _C), lambda i: (i, 0))
    w_spec = pl.BlockSpec((128, 128), lambda i: (0, 0))
    if final:
        grid = ((_N + _R - 1) // _R,)
        out_specs = [pl.BlockSpec((_R, 128), lambda i: (i, 0))]
        out_shape = [jax.ShapeDtypeStruct((_N, 128), jnp.float32)]
    else:
        grid = (_NPAD // _R,)
        out_specs = [slab_spec] * _NS
        out_shape = [jax.ShapeDtypeStruct((_NPAD, _C), jnp.float32)] * _NS
    out = pl.pallas_call(
        body,
        grid=grid,
        in_specs=[part_spec] * _NS + [slab_spec] * _NS + [
            pl.BlockSpec((_R, 1), lambda i: (i, 0)), w_spec, w_spec,
            pl.BlockSpec((1, 128), lambda i: (0, 0)),
        ],
        out_specs=out_specs,
        out_shape=out_shape,
    )(*parts, *slabs, inv, wlt, wrt, b)
    return out[0] if final else out


def kernel(x, edge_index, W1l, W1r, b1, W2l, W2r, b2, W3l, W3r, b3):
    e = edge_index.shape[1]
    src = edge_index[0].astype(jnp.int32)
    dst = edge_index[1].astype(jnp.int32)
    pad = jnp.full((_EPAD - e,), _N, jnp.int32)
    srcb = jnp.concatenate([src, pad]).reshape(_NBTOT, _BATCH)
    dstb = jnp.concatenate([dst, pad]).reshape(_NBTOT, _BATCH)
    edgb = jnp.stack([srcb, dstb], axis=1)

    x_pad = jnp.zeros((_NPAD, _C), jnp.float32)
    x_pad = x_pad.at[:_N, :26].set(x).at[:_N, 26].set(1.0)

    w1lt = jnp.zeros((_C, 128), jnp.float32).at[:26].set(W1l.T)
    w1rt = jnp.zeros((_C, 128), jnp.float32).at[:26].set(W1r.T)

    part1 = _sc_agg(x_pad, edgb)
    *slabs1, inv = _tc_layer1(part1, x_pad, w1lt, w1rt, b1.reshape(1, 128))

    parts2 = [_sc_agg(slabs1[c], edgb) for c in range(_NS)]
    slabs2 = _tc_layer(parts2, slabs1, inv, W2l.T, W2r.T,
                       b2.reshape(1, 128), final=False)

    parts3 = [_sc_agg(slabs2[c], edgb) for c in range(_NS)]
    return _tc_layer(parts3, slabs2, inv, W3l.T, W3r.T,
                     b3.reshape(1, 128), final=True)
